# baseline (speedup 1.0000x reference)
import functools

import jax
import jax.numpy as jnp
from jax import lax
from jax.experimental import pallas as pl
from jax.experimental.pallas import tpu as pltpu
from jax.experimental.pallas import tpu_sc as plsc

_N = 10000
_E = 320000
_D = 128

_NC = 2
_NS = 16
_NW = _NC * _NS
_CH = 128
_CPW = 80
_EPW = _CPW * _CH
_EPAD = _NW * _EPW
_NACC = _N + 128
_NB = 2
_NI = 8
_ROWS_MAIN = 624
_ROWS_LAST = _NACC - 15 * _ROWS_MAIN
_ZROWS = 24


def _agg_body(x_hbm, src_hbm, dst_hbm, out_hbm, acc_sh, *bufs):
    srcs = bufs[:_NI]
    dsts = bufs[_NI:2 * _NI]
    rows = bufs[2 * _NI:2 * _NI + _NB]
    zero_v = bufs[2 * _NI + _NB]
    o = 2 * _NI + _NB + 1
    ssems = bufs[o:o + _NI]
    dsems = bufs[o + _NI:o + 2 * _NI]
    gsems = bufs[o + 2 * _NI:o + 2 * _NI + _NB]
    wsems = bufs[o + 2 * _NI + _NB:]

    c = lax.axis_index("c")
    s = lax.axis_index("s")
    wid = c * _NS + s
    ebase = wid * _EPW

    for b in range(_NI - 2):
        pltpu.async_copy(
            src_hbm.at[pl.ds(ebase + b * _CH, _CH)], srcs[b], ssems[b])
        pltpu.async_copy(
            dst_hbm.at[pl.ds(ebase + b * _CH, _CH)], dsts[b], dsems[b])

    zvec = jnp.zeros((16,), jnp.float32)

    def zrow(i, carry):
        for j in range(8):
            zero_v[i, pl.ds(j * 16, 16)] = zvec
        return carry

    lax.fori_loop(0, _ZROWS, zrow, 0)
    rbase = s * _ROWS_MAIN
    for k in range(_ROWS_MAIN // _ZROWS):
        pltpu.sync_copy(zero_v, acc_sh.at[pl.ds(rbase + k * _ZROWS, _ZROWS)])

    @pl.when(s == _NS - 1)
    def _zero_tail():
        for k in range((_ROWS_LAST - _ROWS_MAIN) // _ZROWS):
            pltpu.sync_copy(
                zero_v,
                acc_sh.at[pl.ds(rbase + _ROWS_MAIN + k * _ZROWS, _ZROWS)])

    plsc.subcore_barrier()

    def group(g, carry):
        for b in range(_NI):
            i = g * _NI + b
            db = b % _NB

            @pl.when(i >= _NB)
            def _drain():
                pltpu.make_async_copy(
                    rows[db], acc_sh.at[dsts[(b - _NB) % _NI]],
                    wsems[db]).wait()

            @pl.when(i + _NI - 2 < _CPW)
            def _loads():
                pltpu.async_copy(
                    src_hbm.at[pl.ds(ebase + (i + _NI - 2) * _CH, _CH)],
                    srcs[(b - 2) % _NI], ssems[(b - 2) % _NI])
                pltpu.async_copy(
                    dst_hbm.at[pl.ds(ebase + (i + _NI - 2) * _CH, _CH)],
                    dsts[(b - 2) % _NI], dsems[(b - 2) % _NI])

            pltpu.make_async_copy(
                src_hbm.at[pl.ds(ebase + i * _CH, _CH)],
                srcs[b], ssems[b]).wait()
            pltpu.async_copy(x_hbm.at[srcs[b]], rows[db], gsems[db])
            pltpu.make_async_copy(
                x_hbm.at[srcs[b]], rows[db], gsems[db]).wait()
            pltpu.make_async_copy(
                dst_hbm.at[pl.ds(ebase + i * _CH, _CH)],
                dsts[b], dsems[b]).wait()
            pltpu.async_copy(rows[db], acc_sh.at[dsts[b]], wsems[db],
                             add=True)
        return carry

    lax.fori_loop(0, _CPW // _NI, group, 0)
    for k in range(_NB):
        i = _CPW - _NB + k
        pltpu.make_async_copy(
            rows[i % _NB], acc_sh.at[dsts[i % _NI]], wsems[i % _NB]).wait()
    plsc.subcore_barrier()

    @pl.when(s < _NS - 1)
    def _copy_main():
        pltpu.sync_copy(acc_sh.at[pl.ds(rbase, _ROWS_MAIN)],
                        out_hbm.at[c, pl.ds(rbase, _ROWS_MAIN)])

    @pl.when(s == _NS - 1)
    def _copy_last():
        pltpu.sync_copy(acc_sh.at[pl.ds(rbase, _ROWS_LAST)],
                        out_hbm.at[c, pl.ds(rbase, _ROWS_LAST)])


_agg_call_cache = []


def _agg_call(x, src, dst):
    if not _agg_call_cache:
        _agg_call_cache.append(functools.partial(
            pl.kernel,
            out_type=jax.ShapeDtypeStruct((_NC, _NACC, _D), jnp.float32),
            mesh=plsc.VectorSubcoreMesh(
                core_axis_name="c", subcore_axis_name="s",
                num_cores=_NC, num_subcores=_NS),
            scratch_types=(
                [pltpu.VMEM_SHARED((_NACC, _D), jnp.float32)]
                + [pltpu.VMEM((_CH,), jnp.int32) for _ in range(2 * _NI)]
                + [pltpu.VMEM((_CH, _D), jnp.float32) for _ in range(_NB)]
                + [pltpu.VMEM((_ZROWS, _D), jnp.float32)]
                + [pltpu.SemaphoreType.DMA for _ in range(2 * _NI + 2 * _NB)]
            ),
        )(_agg_body))
    return _agg_call_cache[0](x, src, dst)


def _mlp_body(x_ref, agg_ref, w1_ref, b1_ref, g_ref, be_ref, w2_ref, b2_ref,
              o_ref):
    h = x_ref[...] + agg_ref[0, :_N, :] + agg_ref[1, :_N, :]
    z = jnp.dot(h, w1_ref[...], preferred_element_type=jnp.float32)
    z = z + b1_ref[...]
    mu = jnp.mean(z, axis=0, keepdims=True)
    d = z - mu
    var = jnp.mean(d * d, axis=0, keepdims=True)
    zn = d * lax.rsqrt(var + 1e-5) * g_ref[...] + be_ref[...]
    zr = jnp.maximum(zn, 0.0)
    o_ref[...] = jnp.dot(zr, w2_ref[...],
                         preferred_element_type=jnp.float32) + b2_ref[...]


def kernel(x, edge_index, W1, b1, gamma, beta, W2, b2):
    pad = _EPAD - _E
    src = jnp.concatenate([edge_index[0], jnp.zeros((pad,), jnp.int32)])
    dst = jnp.concatenate(
        [edge_index[1], _N + jnp.arange(pad, dtype=jnp.int32) % 128])
    agg = _agg_call(x, src, dst)
    hid = W1.shape[1]
    return pl.pallas_call(
        _mlp_body,
        out_shape=jax.ShapeDtypeStruct((x.shape[0], W2.shape[1]), jnp.float32),
    )(x, agg, W1, b1.reshape(1, hid), gamma.reshape(1, hid),
      beta.reshape(1, hid), W2, b2.reshape(1, W2.shape[1]))


# speedup vs baseline: 1.8633x; 1.8633x over previous
import functools

import jax
import jax.numpy as jnp
from jax import lax
from jax.experimental import pallas as pl
from jax.experimental.pallas import tpu as pltpu
from jax.experimental.pallas import tpu_sc as plsc

_N = 10000
_E = 320000
_D = 128

_NC = 2
_NS = 16
_NW = _NC * _NS
_CH = 128
_CHUNKS = _E // _CH
_BASE_CHUNKS = _CHUNKS // _NW
_EXTRA = _CHUNKS - _BASE_CHUNKS * _NW
_ROWS_MAIN = 624
_ROWS_LAST = _N - 15 * _ROWS_MAIN
_ZROWS = 208


def _agg_body(x_hbm, src_hbm, dst_hbm, out_hbm, acc_sh, src_v, dst_v, rows_v,
              zero_v, gsem):
    c = lax.axis_index("c")
    s = lax.axis_index("s")
    wid = c * _NS + s

    zvec = jnp.zeros((16,), jnp.float32)

    def zrow(i, carry):
        for j in range(8):
            zero_v[i, pl.ds(j * 16, 16)] = zvec
        return carry

    lax.fori_loop(0, _ZROWS, zrow, 0)
    rbase = s * _ROWS_MAIN
    for k in range(_ROWS_MAIN // _ZROWS):
        pltpu.sync_copy(zero_v, acc_sh.at[pl.ds(rbase + k * _ZROWS, _ZROWS)])

    @pl.when(s == _NS - 1)
    def _zero_tail():
        pltpu.sync_copy(zero_v.at[pl.ds(0, _ROWS_LAST - _ROWS_MAIN)],
                        acc_sh.at[pl.ds(rbase + _ROWS_MAIN,
                                        _ROWS_LAST - _ROWS_MAIN)])

    plsc.subcore_barrier()

    nchunks = _BASE_CHUNKS + jnp.where(wid < _EXTRA, 1, 0)

    def chunk(j, carry):
        ebase = (wid + j * _NW) * _CH
        pltpu.sync_copy(src_hbm.at[pl.ds(ebase, _CH)], src_v)
        pltpu.sync_copy(dst_hbm.at[pl.ds(ebase, _CH)], dst_v)
        pltpu.async_copy(x_hbm.at[src_v], rows_v, gsem).wait()
        pltpu.sync_copy(rows_v, acc_sh.at[dst_v], add=True)
        return carry

    lax.fori_loop(0, nchunks, chunk, 0)
    plsc.subcore_barrier()

    @pl.when(s < _NS - 1)
    def _copy_main():
        pltpu.sync_copy(acc_sh.at[pl.ds(rbase, _ROWS_MAIN)],
                        out_hbm.at[c, pl.ds(rbase, _ROWS_MAIN)])

    @pl.when(s == _NS - 1)
    def _copy_last():
        pltpu.sync_copy(acc_sh.at[pl.ds(rbase, _ROWS_LAST)],
                        out_hbm.at[c, pl.ds(rbase, _ROWS_LAST)])


_agg_call_cache = []


def _agg_call(x, src, dst):
    if not _agg_call_cache:
        _agg_call_cache.append(functools.partial(
            pl.kernel,
            out_type=jax.ShapeDtypeStruct((_NC, _N, _D), jnp.float32),
            mesh=plsc.VectorSubcoreMesh(
                core_axis_name="c", subcore_axis_name="s",
                num_cores=_NC, num_subcores=_NS),
            scratch_types=[
                pltpu.VMEM_SHARED((_N, _D), jnp.float32),
                pltpu.VMEM((_CH,), jnp.int32),
                pltpu.VMEM((_CH,), jnp.int32),
                pltpu.VMEM((_CH, _D), jnp.float32),
                pltpu.VMEM((_ZROWS, _D), jnp.float32),
                pltpu.SemaphoreType.DMA,
            ],
        )(_agg_body))
    return _agg_call_cache[0](x, src, dst)


def _mlp_body(x_ref, agg_ref, w1_ref, b1_ref, g_ref, be_ref, w2_ref, b2_ref,
              o_ref):
    h = x_ref[...] + agg_ref[0] + agg_ref[1]
    z = jnp.dot(h, w1_ref[...], preferred_element_type=jnp.float32)
    z = z + b1_ref[...]
    mu = jnp.mean(z, axis=0, keepdims=True)
    d = z - mu
    var = jnp.mean(d * d, axis=0, keepdims=True)
    zn = d * lax.rsqrt(var + 1e-5) * g_ref[...] + be_ref[...]
    zr = jnp.maximum(zn, 0.0)
    o_ref[...] = jnp.dot(zr, w2_ref[...],
                         preferred_element_type=jnp.float32) + b2_ref[...]


def kernel(x, edge_index, W1, b1, gamma, beta, W2, b2):
    agg = _agg_call(x, edge_index[0], edge_index[1])
    hid = W1.shape[1]
    return pl.pallas_call(
        _mlp_body,
        out_shape=jax.ShapeDtypeStruct((x.shape[0], W2.shape[1]), jnp.float32),
    )(x, agg, W1, b1.reshape(1, hid), gamma.reshape(1, hid),
      beta.reshape(1, hid), W2, b2.reshape(1, W2.shape[1]))


# speedup vs baseline: 3.0777x; 1.6518x over previous
import functools

import jax
import jax.numpy as jnp
from jax import lax
from jax.experimental import pallas as pl
from jax.experimental.pallas import tpu as pltpu
from jax.experimental.pallas import tpu_sc as plsc

_N = 10000
_E = 320000
_D = 128

_NC = 2
_NS = 16
_NW = _NC * _NS
_CH = 128
_CPW = 80
_EPAD = _NW * _CPW * _CH
_NACC = _N + 128
_NB = 2
_NI = 8
_ROWS_MAIN = 624
_ROWS_LAST = _NACC - 15 * _ROWS_MAIN
_ZROWS = 24


def _agg_body(x_hbm, src_hbm, dst_hbm, out_hbm, acc_sh, *bufs):
    srcs = bufs[:_NI]
    dsts = bufs[_NI:2 * _NI]
    rows = bufs[2 * _NI:2 * _NI + _NB]
    zero_v = bufs[2 * _NI + _NB]
    o = 2 * _NI + _NB + 1
    ssems = bufs[o:o + _NI]
    dsems = bufs[o + _NI:o + 2 * _NI]
    gsems = bufs[o + 2 * _NI:o + 2 * _NI + _NB]
    wsems = bufs[o + 2 * _NI + _NB:]

    c = lax.axis_index("c")
    s = lax.axis_index("s")
    wid = c * _NS + s

    for b in range(_NI - 2):
        pltpu.async_copy(
            src_hbm.at[pl.ds((wid + b * _NW) * _CH, _CH)], srcs[b], ssems[b])
        pltpu.async_copy(
            dst_hbm.at[pl.ds((wid + b * _NW) * _CH, _CH)], dsts[b], dsems[b])

    zvec = jnp.zeros((16,), jnp.float32)

    def zrow(i, carry):
        for j in range(8):
            zero_v[i, pl.ds(j * 16, 16)] = zvec
        return carry

    lax.fori_loop(0, _ZROWS, zrow, 0)
    rbase = s * _ROWS_MAIN
    for k in range(_ROWS_MAIN // _ZROWS):
        pltpu.sync_copy(zero_v, acc_sh.at[pl.ds(rbase + k * _ZROWS, _ZROWS)])

    @pl.when(s == _NS - 1)
    def _zero_tail():
        for k in range((_ROWS_LAST - _ROWS_MAIN) // _ZROWS):
            pltpu.sync_copy(
                zero_v,
                acc_sh.at[pl.ds(rbase + _ROWS_MAIN + k * _ZROWS, _ZROWS)])

    plsc.subcore_barrier()

    def group(g, carry):
        for b in range(_NI):
            i = g * _NI + b
            db = b % _NB

            @pl.when(i >= _NB)
            def _drain():
                pltpu.make_async_copy(
                    rows[db], acc_sh.at[dsts[(b - _NB) % _NI]],
                    wsems[db]).wait()

            @pl.when(i + _NI - 2 < _CPW)
            def _loads():
                pltpu.async_copy(
                    src_hbm.at[pl.ds((wid + (i + _NI - 2) * _NW) * _CH, _CH)],
                    srcs[(b - 2) % _NI], ssems[(b - 2) % _NI])
                pltpu.async_copy(
                    dst_hbm.at[pl.ds((wid + (i + _NI - 2) * _NW) * _CH, _CH)],
                    dsts[(b - 2) % _NI], dsems[(b - 2) % _NI])

            pltpu.make_async_copy(
                src_hbm.at[pl.ds((wid + i * _NW) * _CH, _CH)],
                srcs[b], ssems[b]).wait()
            pltpu.async_copy(x_hbm.at[srcs[b]], rows[db], gsems[db])
            pltpu.make_async_copy(
                x_hbm.at[srcs[b]], rows[db], gsems[db]).wait()
            pltpu.make_async_copy(
                dst_hbm.at[pl.ds((wid + i * _NW) * _CH, _CH)],
                dsts[b], dsems[b]).wait()
            pltpu.async_copy(rows[db], acc_sh.at[dsts[b]], wsems[db],
                             add=True)
        return carry

    lax.fori_loop(0, _CPW // _NI, group, 0)
    for k in range(_NB):
        i = _CPW - _NB + k
        pltpu.make_async_copy(
            rows[i % _NB], acc_sh.at[dsts[i % _NI]], wsems[i % _NB]).wait()
    plsc.subcore_barrier()

    @pl.when(s < _NS - 1)
    def _copy_main():
        pltpu.sync_copy(acc_sh.at[pl.ds(rbase, _ROWS_MAIN)],
                        out_hbm.at[c, pl.ds(rbase, _ROWS_MAIN)])

    @pl.when(s == _NS - 1)
    def _copy_last():
        pltpu.sync_copy(acc_sh.at[pl.ds(rbase, _ROWS_LAST)],
                        out_hbm.at[c, pl.ds(rbase, _ROWS_LAST)])


_agg_call_cache = []


def _agg_call(x, src, dst):
    if not _agg_call_cache:
        _agg_call_cache.append(functools.partial(
            pl.kernel,
            out_type=jax.ShapeDtypeStruct((_NC, _NACC, _D), jnp.float32),
            mesh=plsc.VectorSubcoreMesh(
                core_axis_name="c", subcore_axis_name="s",
                num_cores=_NC, num_subcores=_NS),
            scratch_types=(
                [pltpu.VMEM_SHARED((_NACC, _D), jnp.float32)]
                + [pltpu.VMEM((_CH,), jnp.int32) for _ in range(2 * _NI)]
                + [pltpu.VMEM((_CH, _D), jnp.float32) for _ in range(_NB)]
                + [pltpu.VMEM((_ZROWS, _D), jnp.float32)]
                + [pltpu.SemaphoreType.DMA for _ in range(2 * _NI + 2 * _NB)]
            ),
        )(_agg_body))
    return _agg_call_cache[0](x, src, dst)


def _mlp_body(x_ref, agg_ref, w1_ref, b1_ref, g_ref, be_ref, w2_ref, b2_ref,
              o_ref):
    h = x_ref[...] + agg_ref[0, :_N, :] + agg_ref[1, :_N, :]
    z = jnp.dot(h, w1_ref[...], preferred_element_type=jnp.float32)
    z = z + b1_ref[...]
    mu = jnp.mean(z, axis=0, keepdims=True)
    d = z - mu
    var = jnp.mean(d * d, axis=0, keepdims=True)
    zn = d * lax.rsqrt(var + 1e-5) * g_ref[...] + be_ref[...]
    zr = jnp.maximum(zn, 0.0)
    o_ref[...] = jnp.dot(zr, w2_ref[...],
                         preferred_element_type=jnp.float32) + b2_ref[...]


def kernel(x, edge_index, W1, b1, gamma, beta, W2, b2):
    pad = _EPAD - _E
    padi = jnp.arange(pad, dtype=jnp.int32)
    src = jnp.concatenate([edge_index[0], padi % _N])
    dst = jnp.concatenate([edge_index[1], _N + padi % 128])
    agg = _agg_call(x, src, dst)
    hid = W1.shape[1]
    return pl.pallas_call(
        _mlp_body,
        out_shape=jax.ShapeDtypeStruct((x.shape[0], W2.shape[1]), jnp.float32),
    )(x, agg, W1, b1.reshape(1, hid), gamma.reshape(1, hid),
      beta.reshape(1, hid), W2, b2.reshape(1, W2.shape[1]))


# speedup vs baseline: 3.1028x; 1.0082x over previous
import functools

import jax
import jax.numpy as jnp
from jax import lax
from jax.experimental import pallas as pl
from jax.experimental.pallas import tpu as pltpu
from jax.experimental.pallas import tpu_sc as plsc

_N = 10000
_E = 320000
_D = 128

_NC = 2
_NS = 16
_NW = _NC * _NS
_CH = 128
_CPW = 80
_EPAD = _NW * _CPW * _CH
_NACC = _N + 128
_NB = 2
_NI = 8
_ROWS_MAIN = 624
_ROWS_LAST = _NACC - 15 * _ROWS_MAIN
_ZROWS = 48


def _agg_body(x_hbm, src_hbm, dst_hbm, out_hbm, acc_sh, *bufs):
    srcs = bufs[:_NI]
    dsts = bufs[_NI:2 * _NI]
    rows = bufs[2 * _NI:2 * _NI + _NB]
    zero_v = bufs[2 * _NI + _NB]
    o = 2 * _NI + _NB + 1
    ssems = bufs[o:o + _NI]
    dsems = bufs[o + _NI:o + 2 * _NI]
    gsems = bufs[o + 2 * _NI:o + 2 * _NI + _NB]
    wsems = bufs[o + 2 * _NI + _NB:]

    c = lax.axis_index("c")
    s = lax.axis_index("s")
    wid = c * _NS + s

    for b in range(_NI - 2):
        pltpu.async_copy(
            src_hbm.at[pl.ds((wid + b * _NW) * _CH, _CH)], srcs[b], ssems[b])
        pltpu.async_copy(
            dst_hbm.at[pl.ds((wid + b * _NW) * _CH, _CH)], dsts[b], dsems[b])

    zvec = jnp.zeros((16,), jnp.float32)

    def zrow(i, carry):
        for j in range(8):
            zero_v[i, pl.ds(j * 16, 16)] = zvec
        return carry

    lax.fori_loop(0, _ZROWS, zrow, 0)
    rbase = s * _ROWS_MAIN
    nz = _ROWS_MAIN // _ZROWS
    for k in range(nz):
        pltpu.async_copy(
            zero_v, acc_sh.at[pl.ds(rbase + k * _ZROWS, _ZROWS)],
            wsems[k % _NB])

    @pl.when(s == _NS - 1)
    def _zero_tail():
        for k in range((_ROWS_LAST - _ROWS_MAIN) // _ZROWS):
            pltpu.async_copy(
                zero_v,
                acc_sh.at[pl.ds(rbase + _ROWS_MAIN + k * _ZROWS, _ZROWS)],
                wsems[(nz + k) % _NB])

    for k in range(nz):
        pltpu.make_async_copy(
            zero_v, acc_sh.at[pl.ds(rbase + k * _ZROWS, _ZROWS)],
            wsems[k % _NB]).wait()

    @pl.when(s == _NS - 1)
    def _zero_tail_drain():
        for k in range((_ROWS_LAST - _ROWS_MAIN) // _ZROWS):
            pltpu.make_async_copy(
                zero_v,
                acc_sh.at[pl.ds(rbase + _ROWS_MAIN + k * _ZROWS, _ZROWS)],
                wsems[(nz + k) % _NB]).wait()

    plsc.subcore_barrier()

    def group(g, carry):
        for b in range(_NI):
            i = g * _NI + b
            db = b % _NB

            @pl.when(i >= _NB)
            def _drain():
                pltpu.make_async_copy(
                    rows[db], acc_sh.at[dsts[(b - _NB) % _NI]],
                    wsems[db]).wait()

            @pl.when(i + _NI - 2 < _CPW)
            def _loads():
                pltpu.async_copy(
                    src_hbm.at[pl.ds((wid + (i + _NI - 2) * _NW) * _CH, _CH)],
                    srcs[(b - 2) % _NI], ssems[(b - 2) % _NI])
                pltpu.async_copy(
                    dst_hbm.at[pl.ds((wid + (i + _NI - 2) * _NW) * _CH, _CH)],
                    dsts[(b - 2) % _NI], dsems[(b - 2) % _NI])

            pltpu.make_async_copy(
                src_hbm.at[pl.ds((wid + i * _NW) * _CH, _CH)],
                srcs[b], ssems[b]).wait()
            pltpu.async_copy(x_hbm.at[srcs[b]], rows[db], gsems[db])
            pltpu.make_async_copy(
                x_hbm.at[srcs[b]], rows[db], gsems[db]).wait()
            pltpu.make_async_copy(
                dst_hbm.at[pl.ds((wid + i * _NW) * _CH, _CH)],
                dsts[b], dsems[b]).wait()
            pltpu.async_copy(rows[db], acc_sh.at[dsts[b]], wsems[db],
                             add=True)
        return carry

    lax.fori_loop(0, _CPW // _NI, group, 0)
    for k in range(_NB):
        i = _CPW - _NB + k
        pltpu.make_async_copy(
            rows[i % _NB], acc_sh.at[dsts[i % _NI]], wsems[i % _NB]).wait()
    plsc.subcore_barrier()

    @pl.when(s < _NS - 1)
    def _copy_main():
        pltpu.sync_copy(acc_sh.at[pl.ds(rbase, _ROWS_MAIN)],
                        out_hbm.at[c, pl.ds(rbase, _ROWS_MAIN)])

    @pl.when(s == _NS - 1)
    def _copy_last():
        pltpu.sync_copy(acc_sh.at[pl.ds(rbase, _ROWS_LAST)],
                        out_hbm.at[c, pl.ds(rbase, _ROWS_LAST)])


_agg_call_cache = []


def _agg_call(x, src, dst):
    if not _agg_call_cache:
        _agg_call_cache.append(functools.partial(
            pl.kernel,
            out_type=jax.ShapeDtypeStruct((_NC, _NACC, _D), jnp.float32),
            mesh=plsc.VectorSubcoreMesh(
                core_axis_name="c", subcore_axis_name="s",
                num_cores=_NC, num_subcores=_NS),
            scratch_types=(
                [pltpu.VMEM_SHARED((_NACC, _D), jnp.float32)]
                + [pltpu.VMEM((_CH,), jnp.int32) for _ in range(2 * _NI)]
                + [pltpu.VMEM((_CH, _D), jnp.float32) for _ in range(_NB)]
                + [pltpu.VMEM((_ZROWS, _D), jnp.float32)]
                + [pltpu.SemaphoreType.DMA for _ in range(2 * _NI + 2 * _NB)]
            ),
        )(_agg_body))
    return _agg_call_cache[0](x, src, dst)


def _mlp_body(x_ref, agg_ref, w1_ref, b1_ref, g_ref, be_ref, w2_ref, b2_ref,
              o_ref):
    h = x_ref[...] + agg_ref[0, :_N, :] + agg_ref[1, :_N, :]
    z = jnp.dot(h, w1_ref[...], preferred_element_type=jnp.float32)
    z = z + b1_ref[...]
    mu = jnp.mean(z, axis=0, keepdims=True)
    d = z - mu
    var = jnp.mean(d * d, axis=0, keepdims=True)
    zn = d * lax.rsqrt(var + 1e-5) * g_ref[...] + be_ref[...]
    zr = jnp.maximum(zn, 0.0)
    o_ref[...] = jnp.dot(zr, w2_ref[...],
                         preferred_element_type=jnp.float32) + b2_ref[...]


def kernel(x, edge_index, W1, b1, gamma, beta, W2, b2):
    pad = _EPAD - _E
    padi = jnp.arange(pad, dtype=jnp.int32)
    src = jnp.concatenate([edge_index[0], padi % _N])
    dst = jnp.concatenate([edge_index[1], _N + padi % 128])
    agg = _agg_call(x, src, dst)
    hid = W1.shape[1]
    return pl.pallas_call(
        _mlp_body,
        out_shape=jax.ShapeDtypeStruct((x.shape[0], W2.shape[1]), jnp.float32),
    )(x, agg, W1, b1.reshape(1, hid), gamma.reshape(1, hid),
      beta.reshape(1, hid), W2, b2.reshape(1, W2.shape[1]))
